# baseline (speedup 1.0000x reference)
import functools

import jax
import jax.numpy as jnp
from jax import lax
from jax.experimental import pallas as pl
from jax.experimental.pallas import tpu as pltpu
from jax.experimental.pallas import tpu_sc as plsc

NUM_TOPICS = 8
FT = 32
FS = 16
L = 200
B = 1024
LT = L // 8
BT = B // 128
UNITS = LT * BT


@functools.lru_cache(maxsize=None)
def _make_prep(V):
    info = plsc.get_sparse_core_info()
    NC, NS = info.num_cores, info.num_subcores
    NW = NC * NS
    NCHUNK = V // 128
    VTAIL = V - NCHUNK * 128
    rounds = -(-NCHUNK // NW)

    mesh = plsc.VectorSubcoreMesh(core_axis_name="c", subcore_axis_name="s")

    @functools.partial(
        pl.kernel,
        mesh=mesh,
        out_type=jax.ShapeDtypeStruct((2 * V, 128), jnp.float32),
        compiler_params=pltpu.CompilerParams(
            use_tc_tiling_on_sc=True, needs_layout_passes=False),
        scratch_types=[
            pltpu.VMEM((2, 4, 32, 128), jnp.float32),
            pltpu.VMEM((2, 256, 128), jnp.float32),
            pltpu.SemaphoreType.DMA,
            pltpu.SemaphoreType.DMA,
        ],
    )
    def prep_kernel(tt_hbm, out_hbm, in_v, stage_v, sem_g, sem_o):
        wid = lax.axis_index("s") * NC + lax.axis_index("c")
        iota16 = jnp.arange(16, dtype=jnp.int32)
        rot = [((iota16 + j) & 15) for j in range(16)]

        def fire(chunk, g, cond):
            @pl.when(cond)
            def _():
                for q in range(4):
                    pltpu.async_copy(
                        tt_hbm.at[4 * g + q, pl.ds(0, 32),
                                  pl.ds(chunk * 128, 128)],
                        in_v.at[g, q], sem_g)

        def drain_in(g):
            for q in range(4):
                pltpu.make_async_copy(
                    tt_hbm.at[4 * g + q, pl.ds(0, 32), pl.ds(0, 128)],
                    in_v.at[g, q], sem_g).wait()

        def transpose_g(g, p):
            def body(it, c):
                c0 = (it // 8) * 16
                dv0 = (it % 8) * 16
                qv = jnp.full((16,), c0 // 32, jnp.int32)
                fv = iota16 + (c0 % 32)
                cv = iota16 + c0
                for j in range(16):
                    dvv = rot[j] + dv0
                    x = plsc.load_gather(in_v.at[g], [qv, fv, dvv])
                    plsc.store_scatter(
                        stage_v.at[p], [dvv + dvv + g, cv], x)
                return c
            lax.fori_loop(0, 64, body, 0, unroll=2)

        def drain_out(p):
            pltpu.make_async_copy(
                out_hbm.at[pl.ds(0, 256)], stage_v.at[p], sem_o).wait()

        fire(wid, 0, wid < NCHUNK)
        fire(wid, 1, wid < NCHUNK)

        def chunk_body(r, carry):
            chunk = wid + r * NW
            c_ok = chunk < NCHUNK
            n_ok = chunk + NW < NCHUNK
            p = jnp.bitwise_and(r, 1)

            @pl.when(c_ok)
            def _():
                @pl.when(r >= 2)
                def _():
                    drain_out(p)
                drain_in(0)
                transpose_g(0, p)
                fire(chunk + NW, 0, n_ok)
                drain_in(1)
                transpose_g(1, p)
                fire(chunk + NW, 1, n_ok)
                pltpu.async_copy(
                    stage_v.at[p], out_hbm.at[pl.ds(chunk * 256, 256)],
                    sem_o)
            return carry

        lax.fori_loop(0, rounds, chunk_body, 0)
        for t in (rounds - 2, rounds - 1):
            if t >= 0:
                @pl.when(wid + t * NW < NCHUNK)
                def _(t=t):
                    drain_out(jnp.int32(t & 1))

    return prep_kernel


@functools.lru_cache(maxsize=None)
def _make_kernel(V):
    info = plsc.get_sparse_core_info()
    NC, NS = info.num_cores, info.num_subcores
    NW = NC * NS
    units_per_w = -(-UNITS // NW)

    mesh = plsc.VectorSubcoreMesh(core_axis_name="c", subcore_axis_name="s")
    out_type = tuple(
        jax.ShapeDtypeStruct((FT + FS, LT, BT, 8, 128), jnp.float32)
        for _ in range(NUM_TOPICS)
    )

    VTAIL = V % 128
    VC = V - VTAIL

    @functools.partial(
        pl.kernel,
        mesh=mesh,
        out_type=out_type,
        compiler_params=pltpu.CompilerParams(
            use_tc_tiling_on_sc=False, needs_layout_passes=False),
        scratch_types=[
            pltpu.VMEM((8, 128), jnp.int32),
            pltpu.VMEM((2, 8, 128), jnp.int32),
            pltpu.VMEM((2, 128, 128), jnp.float32),
            pltpu.VMEM((2, 4, FT, 128), jnp.float32),
            pltpu.VMEM((2, 128, FS), jnp.float32),
            pltpu.VMEM((2, FS, 128), jnp.float32),
            pltpu.VMEM((2 * max(VTAIL, 1), 128), jnp.float32),
            pltpu.SemaphoreType.DMA,
            pltpu.SemaphoreType.DMA,
        ],
    )
    def topic_kernel(seq_hbm, topics_hbm, shared_hbm, tail_hbm, *rest):
        outs = rest[:NUM_TOPICS]
        (idx_v, idx2_v, rows_v, t_v, sh_rows_v, sh_t_v, tail_v,
         sem_g, sem_o) = rest[NUM_TOPICS:]
        pltpu.sync_copy(tail_hbm, tail_v)

        wid = lax.axis_index("s") * NC + lax.axis_index("c")
        iota16 = jnp.arange(16, dtype=jnp.int32)
        rot = [((iota16 + j) & 15) for j in range(16)]

        steps = [(g, l8) for l8 in range(8) for g in range(2)]
        steps += [(2, l8) for l8 in range(8)]

        def transpose_topics(rows_ref, t_ref):
            def body(it, c):
                cb = lax.shift_right_logical(it, 3)
                b0 = lax.shift_left(jnp.bitwise_and(it, 7), 4)
                c0 = lax.shift_left(cb, 4)
                cv = iota16 + c0
                fv = iota16 + jnp.bitwise_and(c0, 31)
                qv = jnp.full((16,), lax.shift_right_logical(cb, 1), jnp.int32)
                for j in range(16):
                    bv = rot[j] + b0
                    x = plsc.load_gather(rows_ref, [bv, cv])
                    plsc.store_scatter(t_ref, [qv, fv, bv], x)
                return c
            lax.fori_loop(0, 64, body, 0, unroll=2)

        def transpose_shared(rows_ref, t_ref):
            def body(it, c):
                b0 = it * 16
                for j in range(16):
                    bv = rot[j] + b0
                    x = plsc.load_gather(rows_ref, [bv, iota16])
                    plsc.store_scatter(t_ref, [iota16, bv], x)
                return c
            lax.fori_loop(0, 8, body, 0, unroll=2)

        def unit_body(r, carry):
            u = wid + r * NW
            valid = u < UNITS

            @pl.when(valid)
            def _():
                ult = u // BT
                ubt = u - ult * BT
                pltpu.sync_copy(seq_hbm.at[ult, ubt], idx_v)

                def idx_body(it, c):
                    l8 = it // 8
                    c0 = (it - l8 * 8) * 16
                    tok = idx_v[l8, pl.ds(c0, 16)]
                    d = tok + tok
                    idx2_v[0, l8, pl.ds(c0, 16)] = d
                    idx2_v[1, l8, pl.ds(c0, 16)] = d + 1
                    return c
                lax.fori_loop(0, 64, idx_body, 0, unroll=2)

                def fire(s):
                    g, l8 = steps[s]
                    p = s & 1
                    if g < 2:
                        return [pltpu.async_copy(
                            topics_hbm.at[idx2_v.at[g, l8]],
                            rows_v.at[p], sem_g)]
                    return [pltpu.async_copy(
                        shared_hbm.at[idx_v.at[l8]],
                        sh_rows_v.at[p], sem_g)]

                pending_g = fire(0)
                pending_o = [[], []]
                for s in range(len(steps)):
                    g, l8 = steps[s]
                    p = s & 1
                    nxt = fire(s + 1) if s + 1 < len(steps) else []
                    for c in pending_g:
                        c.wait()
                    for c in pending_o[p]:
                        c.wait()
                    if g < 2:
                        def fix_body(blk, c, p=p, g=g, l8=l8):
                            tokv = idx_v[l8, pl.ds(blk * 16, 16)]

                            @pl.when(jnp.max(tokv) >= VC)
                            def _():
                                m = tokv >= VC
                                dvv = jnp.maximum(tokv - VC, 0)
                                rowv = dvv + dvv + g
                                bv = iota16 + blk * 16

                                def col_body(col, cc):
                                    cv = jnp.full((16,), col, jnp.int32)
                                    cur = plsc.load_gather(
                                        rows_v.at[p], [bv, cv])
                                    t = plsc.load_gather(
                                        tail_v, [rowv, cv])
                                    plsc.store_scatter(
                                        rows_v.at[p], [bv, cv],
                                        jnp.where(m, t, cur))
                                    return cc
                                lax.fori_loop(0, 128, col_body, 0)
                            return c
                        lax.fori_loop(0, 8, fix_body, 0)
                        transpose_topics(rows_v.at[p], t_v.at[p])
                        pending_o[p] = [pltpu.async_copy(
                            t_v.at[p, q],
                            outs[4 * g + q].at[pl.ds(0, FT), ult, ubt, l8],
                            sem_o) for q in range(4)]
                    else:
                        transpose_shared(sh_rows_v.at[p], sh_t_v.at[p])
                        pending_o[p] = [pltpu.async_copy(
                            sh_t_v.at[p],
                            outs[k].at[pl.ds(FT, FS), ult, ubt, l8],
                            sem_o) for k in range(NUM_TOPICS)]
                    pending_g = nxt
                for ps in pending_o:
                    for c in ps:
                        c.wait()
            return carry

        lax.fori_loop(0, units_per_w, unit_body, 0)

    return topic_kernel


def kernel(sequence, topic_tables, shared_table):
    V = topic_tables.shape[1]
    seq4 = (sequence.astype(jnp.int32)
            .reshape(BT, 128, LT, 8).transpose(2, 0, 3, 1))
    ttv = topic_tables.transpose(0, 2, 1)
    tt = _make_prep(V)(ttv)
    vtail = V % 128
    vc = V - vtail
    tail = (topic_tables[:, vc:, :].transpose(1, 0, 2)
            .reshape(2 * max(vtail, 1), (NUM_TOPICS // 2) * FT))
    outs = _make_kernel(V)(seq4, tt, shared_table, tail)
    return tuple(
        o.transpose(2, 4, 0, 1, 3).reshape(B, FT + FS, L) for o in outs
    )


# speedup vs baseline: 1.5345x; 1.5345x over previous
import functools

import jax
import jax.numpy as jnp
from jax import lax
from jax.experimental import pallas as pl
from jax.experimental.pallas import tpu as pltpu
from jax.experimental.pallas import tpu_sc as plsc

NUM_TOPICS = 8
FT = 32
FS = 16
L = 200
B = 1024
LT = L // 8
BT = B // 128
UNITS = LT * BT


@functools.lru_cache(maxsize=None)
def _make_prep(V):
    info = plsc.get_sparse_core_info()
    NC, NS = info.num_cores, info.num_subcores
    NW = NC * NS
    NCHUNK = V // 128
    VTAIL = V - NCHUNK * 128
    rounds = -(-NCHUNK // NW)

    mesh = plsc.VectorSubcoreMesh(core_axis_name="c", subcore_axis_name="s")

    @functools.partial(
        pl.kernel,
        mesh=mesh,
        out_type=jax.ShapeDtypeStruct((2 * V, 128), jnp.float32),
        compiler_params=pltpu.CompilerParams(
            use_tc_tiling_on_sc=True, needs_layout_passes=False),
        scratch_types=[
            pltpu.VMEM((2, 4, 32, 128), jnp.float32),
            pltpu.VMEM((2, 256, 128), jnp.float32),
            pltpu.SemaphoreType.DMA,
            pltpu.SemaphoreType.DMA,
        ],
    )
    def prep_kernel(tt_hbm, out_hbm, in_v, stage_v, sem_g, sem_o):
        wid = lax.axis_index("s") * NC + lax.axis_index("c")
        iota16 = jnp.arange(16, dtype=jnp.int32)
        rot = [((iota16 + j) & 15) for j in range(16)]

        def fire(chunk, g, cond):
            @pl.when(cond)
            def _():
                for q in range(4):
                    pltpu.async_copy(
                        tt_hbm.at[4 * g + q, pl.ds(0, 32),
                                  pl.ds(chunk * 128, 128)],
                        in_v.at[g, q], sem_g)

        def drain_in(g):
            for q in range(4):
                pltpu.make_async_copy(
                    tt_hbm.at[4 * g + q, pl.ds(0, 32), pl.ds(0, 128)],
                    in_v.at[g, q], sem_g).wait()

        def transpose_g(g, p):
            def body(it, c):
                c0 = (it // 8) * 16
                dv0 = (it % 8) * 16
                qv = jnp.full((16,), c0 // 32, jnp.int32)
                fv = iota16 + (c0 % 32)
                cv = iota16 + c0
                for j in range(16):
                    dvv = rot[j] + dv0
                    x = plsc.load_gather(in_v.at[g], [qv, fv, dvv])
                    plsc.store_scatter(
                        stage_v.at[p], [dvv + dvv + g, cv], x)
                return c
            lax.fori_loop(0, 64, body, 0, unroll=2)

        def drain_out(p):
            pltpu.make_async_copy(
                out_hbm.at[pl.ds(0, 256)], stage_v.at[p], sem_o).wait()

        fire(wid, 0, wid < NCHUNK)
        fire(wid, 1, wid < NCHUNK)

        def chunk_body(r, carry):
            chunk = wid + r * NW
            c_ok = chunk < NCHUNK
            n_ok = chunk + NW < NCHUNK
            p = jnp.bitwise_and(r, 1)

            @pl.when(c_ok)
            def _():
                @pl.when(r >= 2)
                def _():
                    drain_out(p)
                drain_in(0)
                transpose_g(0, p)
                fire(chunk + NW, 0, n_ok)
                drain_in(1)
                transpose_g(1, p)
                fire(chunk + NW, 1, n_ok)
                pltpu.async_copy(
                    stage_v.at[p], out_hbm.at[pl.ds(chunk * 256, 256)],
                    sem_o)
            return carry

        lax.fori_loop(0, rounds, chunk_body, 0)
        for t in (rounds - 2, rounds - 1):
            if t >= 0:
                @pl.when(wid + t * NW < NCHUNK)
                def _(t=t):
                    drain_out(jnp.int32(t & 1))

    return prep_kernel


@functools.lru_cache(maxsize=None)
def _make_kernel(V):
    info = plsc.get_sparse_core_info()
    NC, NS = info.num_cores, info.num_subcores
    NW = NC * NS
    units_per_w = -(-UNITS // NW)

    mesh = plsc.VectorSubcoreMesh(core_axis_name="c", subcore_axis_name="s")
    out_type = tuple(
        jax.ShapeDtypeStruct((FT + FS, LT, BT, 8, 128), jnp.float32)
        for _ in range(NUM_TOPICS)
    )

    VTAIL = V % 128
    VC = V - VTAIL

    @functools.partial(
        pl.kernel,
        mesh=mesh,
        out_type=out_type,
        compiler_params=pltpu.CompilerParams(
            use_tc_tiling_on_sc=False, needs_layout_passes=False),
        scratch_types=[
            pltpu.VMEM((8, 128), jnp.int32),
            pltpu.VMEM((2, 8, 128), jnp.int32),
            pltpu.VMEM((2, 128, 128), jnp.float32),
            pltpu.VMEM((2, 4, FT, 128), jnp.float32),
            pltpu.VMEM((2, 128, FS), jnp.float32),
            pltpu.VMEM((2, FS, 128), jnp.float32),
            pltpu.VMEM((2 * max(VTAIL, 1), 128), jnp.float32),
            pltpu.SemaphoreType.DMA,
            pltpu.SemaphoreType.DMA,
        ],
    )
    def topic_kernel(seq_hbm, topics_hbm, shared_hbm, tail_hbm, *rest):
        outs = rest[:NUM_TOPICS]
        (idx_v, idx2_v, rows_v, t_v, sh_rows_v, sh_t_v, tail_v,
         sem_g, sem_o) = rest[NUM_TOPICS:]
        pltpu.sync_copy(tail_hbm, tail_v)

        wid = lax.axis_index("s") * NC + lax.axis_index("c")
        iota16 = jnp.arange(16, dtype=jnp.int32)
        rot = [((iota16 + j) & 15) for j in range(16)]

        steps = [(g, l8) for l8 in range(8) for g in range(2)]
        steps += [(2, l8) for l8 in range(8)]

        def transpose_topics(rows_ref, t_ref):
            def body(it, c):
                cb = lax.shift_right_logical(it, 3)
                b0 = lax.shift_left(jnp.bitwise_and(it, 7), 4)
                c0 = lax.shift_left(cb, 4)
                cv = iota16 + c0
                fv = iota16 + jnp.bitwise_and(c0, 31)
                qv = jnp.full((16,), lax.shift_right_logical(cb, 1), jnp.int32)
                for j in range(16):
                    bv = rot[j] + b0
                    x = plsc.load_gather(rows_ref, [bv, cv])
                    plsc.store_scatter(t_ref, [qv, fv, bv], x)
                return c

            plsc.parallel_loop(0, 64, unroll=2, carry=jnp.int32(0))(body)

        def transpose_shared(rows_ref, t_ref):
            def body(it, c):
                b0 = it * 16
                for j in range(16):
                    bv = rot[j] + b0
                    x = plsc.load_gather(rows_ref, [bv, iota16])
                    plsc.store_scatter(t_ref, [iota16, bv], x)
                return c
            lax.fori_loop(0, 8, body, 0, unroll=2)

        def unit_body(r, carry):
            u = wid + r * NW
            valid = u < UNITS

            @pl.when(valid)
            def _():
                ult = u // BT
                ubt = u - ult * BT
                pltpu.sync_copy(seq_hbm.at[ult, ubt], idx_v)

                def idx_body(it, c):
                    l8 = it // 8
                    c0 = (it - l8 * 8) * 16
                    tok = idx_v[l8, pl.ds(c0, 16)]
                    d = tok + tok
                    idx2_v[0, l8, pl.ds(c0, 16)] = d
                    idx2_v[1, l8, pl.ds(c0, 16)] = d + 1
                    return c
                lax.fori_loop(0, 64, idx_body, 0, unroll=2)

                def fire(s):
                    g, l8 = steps[s]
                    p = s & 1
                    if g < 2:
                        return [pltpu.async_copy(
                            topics_hbm.at[idx2_v.at[g, l8]],
                            rows_v.at[p], sem_g)]
                    return [pltpu.async_copy(
                        shared_hbm.at[idx_v.at[l8]],
                        sh_rows_v.at[p], sem_g)]

                pending_g = fire(0)
                pending_o = [[], []]
                for s in range(len(steps)):
                    g, l8 = steps[s]
                    p = s & 1
                    nxt = fire(s + 1) if s + 1 < len(steps) else []
                    for c in pending_g:
                        c.wait()
                    for c in pending_o[p]:
                        c.wait()
                    if g < 2:
                        def fix_body(blk, c, p=p, g=g, l8=l8):
                            tokv = idx_v[l8, pl.ds(blk * 16, 16)]

                            @pl.when(jnp.max(tokv) >= VC)
                            def _():
                                m = tokv >= VC
                                dvv = jnp.maximum(tokv - VC, 0)
                                rowv = dvv + dvv + g
                                bv = iota16 + blk * 16

                                def col_body(col, cc):
                                    cv = jnp.full((16,), col, jnp.int32)
                                    cur = plsc.load_gather(
                                        rows_v.at[p], [bv, cv])
                                    t = plsc.load_gather(
                                        tail_v, [rowv, cv])
                                    plsc.store_scatter(
                                        rows_v.at[p], [bv, cv],
                                        jnp.where(m, t, cur))
                                    return cc
                                lax.fori_loop(0, 128, col_body, 0)
                            return c
                        lax.fori_loop(0, 8, fix_body, 0)
                        transpose_topics(rows_v.at[p], t_v.at[p])
                        pending_o[p] = [pltpu.async_copy(
                            t_v.at[p, q],
                            outs[4 * g + q].at[pl.ds(0, FT), ult, ubt, l8],
                            sem_o) for q in range(4)]
                    else:
                        transpose_shared(sh_rows_v.at[p], sh_t_v.at[p])
                        pending_o[p] = [pltpu.async_copy(
                            sh_t_v.at[p],
                            outs[k].at[pl.ds(FT, FS), ult, ubt, l8],
                            sem_o) for k in range(NUM_TOPICS)]
                    pending_g = nxt
                for ps in pending_o:
                    for c in ps:
                        c.wait()
            return carry

        lax.fori_loop(0, units_per_w, unit_body, 0)

    return topic_kernel


def kernel(sequence, topic_tables, shared_table):
    V = topic_tables.shape[1]
    seq4 = (sequence.astype(jnp.int32)
            .reshape(BT, 128, LT, 8).transpose(2, 0, 3, 1))
    ttv = topic_tables.transpose(0, 2, 1)
    tt = _make_prep(V)(ttv)
    vtail = V % 128
    vc = V - vtail
    tail = (topic_tables[:, vc:, :].transpose(1, 0, 2)
            .reshape(2 * max(vtail, 1), (NUM_TOPICS // 2) * FT))
    outs = _make_kernel(V)(seq4, tt, shared_table, tail)
    return tuple(
        o.transpose(2, 4, 0, 1, 3).reshape(B, FT + FS, L) for o in outs
    )


# speedup vs baseline: 2.0098x; 1.3098x over previous
import functools

import jax
import jax.numpy as jnp
from jax import lax
from jax.experimental import pallas as pl
from jax.experimental.pallas import tpu as pltpu
from jax.experimental.pallas import tpu_sc as plsc

NUM_TOPICS = 8
FT = 32
FS = 16
L = 200
B = 1024
LT = L // 8
BT = B // 128
UNITS = LT * BT


@functools.lru_cache(maxsize=None)
def _make_prep(V):
    info = plsc.get_sparse_core_info()
    NC, NS = info.num_cores, info.num_subcores
    NW = NC * NS
    NCHUNK = V // 128
    VTAIL = V - NCHUNK * 128
    rounds = -(-NCHUNK // NW)

    mesh = plsc.VectorSubcoreMesh(core_axis_name="c", subcore_axis_name="s")

    @functools.partial(
        pl.kernel,
        mesh=mesh,
        out_type=jax.ShapeDtypeStruct((2 * V, 128), jnp.float32),
        compiler_params=pltpu.CompilerParams(
            use_tc_tiling_on_sc=True, needs_layout_passes=False),
        scratch_types=[
            pltpu.VMEM((2, 4, 32, 128), jnp.float32),
            pltpu.VMEM((2, 256, 128), jnp.float32),
            pltpu.SemaphoreType.DMA,
            pltpu.SemaphoreType.DMA,
        ],
    )
    def prep_kernel(tt_hbm, out_hbm, in_v, stage_v, sem_g, sem_o):
        wid = lax.axis_index("s") * NC + lax.axis_index("c")
        iota16 = jnp.arange(16, dtype=jnp.int32)
        rot = [((iota16 + j) & 15) for j in range(16)]

        def fire(chunk, g, cond):
            @pl.when(cond)
            def _():
                for q in range(4):
                    pltpu.async_copy(
                        tt_hbm.at[4 * g + q, pl.ds(0, 32),
                                  pl.ds(chunk * 128, 128)],
                        in_v.at[g, q], sem_g)

        def drain_in(g):
            for q in range(4):
                pltpu.make_async_copy(
                    tt_hbm.at[4 * g + q, pl.ds(0, 32), pl.ds(0, 128)],
                    in_v.at[g, q], sem_g).wait()

        def transpose_g(g, p):
            def body(it, c):
                c0 = (it // 8) * 16
                dv0 = (it % 8) * 16
                qv = jnp.full((16,), c0 // 32, jnp.int32)
                fv = iota16 + (c0 % 32)
                cv = iota16 + c0
                for j in range(16):
                    dvv = rot[j] + dv0
                    x = plsc.load_gather(in_v.at[g], [qv, fv, dvv])
                    plsc.store_scatter(
                        stage_v.at[p], [dvv + dvv + g, cv], x)
                return c

            plsc.parallel_loop(0, 64, unroll=2, carry=jnp.int32(0))(body)

        def drain_out(p):
            pltpu.make_async_copy(
                out_hbm.at[pl.ds(0, 256)], stage_v.at[p], sem_o).wait()

        fire(wid, 0, wid < NCHUNK)
        fire(wid, 1, wid < NCHUNK)

        def chunk_body(r, carry):
            chunk = wid + r * NW
            c_ok = chunk < NCHUNK
            n_ok = chunk + NW < NCHUNK
            p = jnp.bitwise_and(r, 1)

            @pl.when(c_ok)
            def _():
                @pl.when(r >= 2)
                def _():
                    drain_out(p)
                drain_in(0)
                transpose_g(0, p)
                fire(chunk + NW, 0, n_ok)
                drain_in(1)
                transpose_g(1, p)
                fire(chunk + NW, 1, n_ok)
                pltpu.async_copy(
                    stage_v.at[p], out_hbm.at[pl.ds(chunk * 256, 256)],
                    sem_o)
            return carry

        lax.fori_loop(0, rounds, chunk_body, 0)
        for t in (rounds - 2, rounds - 1):
            if t >= 0:
                @pl.when(wid + t * NW < NCHUNK)
                def _(t=t):
                    drain_out(jnp.int32(t & 1))

    return prep_kernel


@functools.lru_cache(maxsize=None)
def _make_kernel(V):
    info = plsc.get_sparse_core_info()
    NC, NS = info.num_cores, info.num_subcores
    NW = NC * NS
    units_per_w = -(-UNITS // NW)

    mesh = plsc.VectorSubcoreMesh(core_axis_name="c", subcore_axis_name="s")
    out_type = tuple(
        jax.ShapeDtypeStruct((FT + FS, LT, BT, 8, 128), jnp.float32)
        for _ in range(NUM_TOPICS)
    )

    VTAIL = V % 128
    VC = V - VTAIL

    @functools.partial(
        pl.kernel,
        mesh=mesh,
        out_type=out_type,
        compiler_params=pltpu.CompilerParams(
            use_tc_tiling_on_sc=False, needs_layout_passes=False),
        scratch_types=[
            pltpu.VMEM((8, 128), jnp.int32),
            pltpu.VMEM((2, 8, 128), jnp.int32),
            pltpu.VMEM((2, 128, 128), jnp.float32),
            pltpu.VMEM((2, 4, FT, 128), jnp.float32),
            pltpu.VMEM((2, 128, FS), jnp.float32),
            pltpu.VMEM((2, FS, 128), jnp.float32),
            pltpu.VMEM((2 * max(VTAIL, 1), 128), jnp.float32),
            pltpu.SemaphoreType.DMA,
            pltpu.SemaphoreType.DMA,
        ],
    )
    def topic_kernel(seq_hbm, topics_hbm, shared_hbm, tail_hbm, *rest):
        outs = rest[:NUM_TOPICS]
        (idx_v, idx2_v, rows_v, t_v, sh_rows_v, sh_t_v, tail_v,
         sem_g, sem_o) = rest[NUM_TOPICS:]
        pltpu.sync_copy(tail_hbm, tail_v)

        wid = lax.axis_index("s") * NC + lax.axis_index("c")
        iota16 = jnp.arange(16, dtype=jnp.int32)
        rot = [((iota16 + j) & 15) for j in range(16)]

        steps = [(g, l8) for l8 in range(8) for g in range(2)]
        steps += [(2, l8) for l8 in range(8)]

        def transpose_topics(rows_ref, t_ref):
            def body(it, c):
                cb = lax.shift_right_logical(it, 3)
                b0 = lax.shift_left(jnp.bitwise_and(it, 7), 4)
                c0 = lax.shift_left(cb, 4)
                cv = iota16 + c0
                fv = iota16 + jnp.bitwise_and(c0, 31)
                qv = jnp.full((16,), lax.shift_right_logical(cb, 1), jnp.int32)
                for j in range(16):
                    bv = rot[j] + b0
                    x = plsc.load_gather(rows_ref, [bv, cv])
                    plsc.store_scatter(t_ref, [qv, fv, bv], x)
                return c

            plsc.parallel_loop(0, 64, unroll=2, carry=jnp.int32(0))(body)

        def transpose_shared(rows_ref, t_ref):
            def body(it, c):
                b0 = it * 16
                for j in range(16):
                    bv = rot[j] + b0
                    x = plsc.load_gather(rows_ref, [bv, iota16])
                    plsc.store_scatter(t_ref, [iota16, bv], x)
                return c

            plsc.parallel_loop(0, 8, unroll=2, carry=jnp.int32(0))(body)

        def unit_body(r, carry):
            u = wid + r * NW
            valid = u < UNITS

            @pl.when(valid)
            def _():
                ult = u // BT
                ubt = u - ult * BT
                pltpu.sync_copy(seq_hbm.at[ult, ubt], idx_v)

                def idx_body(it, c):
                    l8 = it // 8
                    c0 = (it - l8 * 8) * 16
                    tok = idx_v[l8, pl.ds(c0, 16)]
                    d = tok + tok
                    idx2_v[0, l8, pl.ds(c0, 16)] = d
                    idx2_v[1, l8, pl.ds(c0, 16)] = d + 1
                    return c

                plsc.parallel_loop(0, 64, unroll=2,
                                   carry=jnp.int32(0))(idx_body)

                def fire(s):
                    g, l8 = steps[s]
                    p = s & 1
                    if g < 2:
                        return [pltpu.async_copy(
                            topics_hbm.at[idx2_v.at[g, l8]],
                            rows_v.at[p], sem_g)]
                    return [pltpu.async_copy(
                        shared_hbm.at[idx_v.at[l8]],
                        sh_rows_v.at[p], sem_g)]

                pending_g = fire(0)
                pending_o = [[], []]
                for s in range(len(steps)):
                    g, l8 = steps[s]
                    p = s & 1
                    nxt = fire(s + 1) if s + 1 < len(steps) else []
                    for c in pending_g:
                        c.wait()
                    for c in pending_o[p]:
                        c.wait()
                    if g < 2:
                        def fix_body(blk, c, p=p, g=g, l8=l8):
                            tokv = idx_v[l8, pl.ds(blk * 16, 16)]

                            @pl.when(jnp.max(tokv) >= VC)
                            def _():
                                m = tokv >= VC
                                dvv = jnp.maximum(tokv - VC, 0)
                                rowv = dvv + dvv + g
                                bv = iota16 + blk * 16

                                def col_body(col, cc):
                                    cv = jnp.full((16,), col, jnp.int32)
                                    cur = plsc.load_gather(
                                        rows_v.at[p], [bv, cv])
                                    t = plsc.load_gather(
                                        tail_v, [rowv, cv])
                                    plsc.store_scatter(
                                        rows_v.at[p], [bv, cv],
                                        jnp.where(m, t, cur))
                                    return cc
                                lax.fori_loop(0, 128, col_body, 0)
                            return c
                        lax.fori_loop(0, 8, fix_body, 0)
                        transpose_topics(rows_v.at[p], t_v.at[p])
                        pending_o[p] = [pltpu.async_copy(
                            t_v.at[p, q],
                            outs[4 * g + q].at[pl.ds(0, FT), ult, ubt, l8],
                            sem_o) for q in range(4)]
                    else:
                        transpose_shared(sh_rows_v.at[p], sh_t_v.at[p])
                        pending_o[p] = [pltpu.async_copy(
                            sh_t_v.at[p],
                            outs[k].at[pl.ds(FT, FS), ult, ubt, l8],
                            sem_o) for k in range(NUM_TOPICS)]
                    pending_g = nxt
                for ps in pending_o:
                    for c in ps:
                        c.wait()
            return carry

        lax.fori_loop(0, units_per_w, unit_body, 0)

    return topic_kernel


def kernel(sequence, topic_tables, shared_table):
    V = topic_tables.shape[1]
    seq4 = (sequence.astype(jnp.int32)
            .reshape(BT, 128, LT, 8).transpose(2, 0, 3, 1))
    ttv = topic_tables.transpose(0, 2, 1)
    tt = _make_prep(V)(ttv)
    vtail = V % 128
    vc = V - vtail
    tail = (topic_tables[:, vc:, :].transpose(1, 0, 2)
            .reshape(2 * max(vtail, 1), (NUM_TOPICS // 2) * FT))
    outs = _make_kernel(V)(seq4, tt, shared_table, tail)
    return tuple(
        o.transpose(2, 4, 0, 1, 3).reshape(B, FT + FS, L) for o in outs
    )
